# baseline (speedup 1.0000x reference)
import functools

import jax
import jax.numpy as jnp
from jax import lax
from jax.experimental import pallas as pl
from jax.experimental.pallas import tpu as pltpu
from jax.experimental.pallas import tpu_sc as plsc

N = 10000
E = 160000
F = 256
SLOPE = 0.2
NC, NS, L = 2, 16, 16
NW = NC * NS
NPAD = 10240
NEG = -1e30
NB = 6
CSZ = 2048
EPW = E // NW
COLPAD = E + CSZ + 8


def _mm_body(x_ref, w_ref, al_ref, ar_ref, h_ref, el_ref, er_ref):
    h = jnp.dot(x_ref[...], w_ref[...], preferred_element_type=jnp.float32)
    h_ref[...] = h
    el_ref[...] = jnp.dot(h, al_ref[...], preferred_element_type=jnp.float32)
    er_ref[...] = jnp.dot(h, ar_ref[...], preferred_element_type=jnp.float32)


def _dense_stage(x, W, al, ar):
    BM = 256
    h, el, er = pl.pallas_call(
        _mm_body,
        grid=(NPAD // BM,),
        in_specs=[
            pl.BlockSpec((BM, F), lambda i: (i, 0)),
            pl.BlockSpec((F, F), lambda i: (0, 0)),
            pl.BlockSpec((F, 1), lambda i: (0, 0)),
            pl.BlockSpec((F, 1), lambda i: (0, 0)),
        ],
        out_specs=[
            pl.BlockSpec((BM, F), lambda i: (i, 0)),
            pl.BlockSpec((BM, 1), lambda i: (i, 0)),
            pl.BlockSpec((BM, 1), lambda i: (i, 0)),
        ],
        out_shape=[
            jax.ShapeDtypeStruct((NPAD, F), jnp.float32),
            jax.ShapeDtypeStruct((NPAD, 1), jnp.float32),
            jax.ShapeDtypeStruct((NPAD, 1), jnp.float32),
        ],
    )(x, W, al.reshape(F, 1), ar.reshape(F, 1))
    return h, el.reshape(NPAD), er.reshape(NPAD)


def _gat_edge_body(rowp_hbm, cidx_hbm, el_hbm, er_hbm, h_hbm, bias_hbm,
                   out_hbm, rowp_v, el_v, er_v, bias_v, cidx_v, rows_v,
                   erring_v, orow_v, sem, osem):
    c = lax.axis_index("c")
    s = lax.axis_index("s")
    wid = s * NC + c

    pltpu.sync_copy(rowp_hbm, rowp_v)
    pltpu.sync_copy(el_hbm, el_v.at[pl.ds(0, NPAD)])
    pltpu.sync_copy(er_hbm, er_v)
    pltpu.sync_copy(bias_hbm, bias_v)

    lanes = lax.iota(jnp.int32, 16)

    def rp_at(i):
        return rowp_v[pl.ds(i, 16)][0]

    def bsearch(target):
        def bb(_, lohi):
            lo, hi = lohi
            run = lo < hi
            mid = (lo + hi) // 2
            go_hi = rp_at(mid) >= target
            nlo = jnp.where(go_hi, lo, mid + 1)
            nhi = jnp.where(go_hi, mid, hi)
            return (jnp.where(run, nlo, lo), jnp.where(run, nhi, hi))
        lo, _ = lax.fori_loop(0, 15, bb, (jnp.int32(0), jnp.int32(N)))
        return lo

    rlo = bsearch(wid * EPW)
    rhi = jnp.where(wid == NW - 1, jnp.int32(N), bsearch((wid + 1) * EPW))

    def skip_empty(row, nb):
        def cond(carry):
            return carry[1] != 0
        def body(carry):
            r, _ = carry
            adv = (rp_at(r + 1) <= nb) & (r + 1 < rhi)
            return (r + jnp.where(adv, 1, 0), adv.astype(jnp.int32))
        r, _ = lax.while_loop(cond, body, (row, jnp.int32(1)))
        return r

    def issue(buf, la_base, cb):
        need = la_base + 16 > cb + CSZ
        nstart = pl.multiple_of((la_base // 8) * 8, 8)
        ncb = jnp.where(need, nstart, cb)

        @pl.when(need)
        def _():
            pltpu.sync_copy(cidx_hbm.at[pl.ds(nstart, CSZ)], cidx_v)

        nidx = cidx_v[pl.ds(la_base - ncb, 16)]
        erring_v[buf, :] = plsc.load_gather(er_v, [nidx])
        pltpu.async_copy(h_hbm.at[nidx], rows_v.at[buf], sem.at[buf])
        return ncb

    def advance(la_row, la_base):
        e1la = rp_at(la_row + 1)
        nb = jnp.minimum(la_base + 16, e1la)
        nrow = jnp.where(nb < e1la, la_row, skip_empty(la_row, nb))
        return nrow, nb

    la_base = rp_at(rlo)
    la_row = skip_empty(rlo, la_base)
    cb = jnp.int32(-(CSZ + 64))
    for i in range(NB):
        cb = issue(jnp.int32(i), la_base, cb)
        la_row, la_base = advance(la_row, la_base)

    def row_body(r, rcarry):
        q, cb, la_row, la_base, ro = rcarry
        rp = rowp_v[pl.ds(r, 16)]
        e0 = rp[0]
        e1 = rp[1]
        deg = e1 - e0
        ngrp = (deg + 15) // 16
        el_r = jnp.full((16,), el_v[pl.ds(r, 16)][0], jnp.float32)

        def grp_body(g, carry):
            q, cb, la_row, la_base, m_v, denom_v, acc = carry
            buf = q % NB
            erg = erring_v[buf, :]
            sc = el_r + erg
            sc = jnp.where(sc > 0, sc, SLOPE * sc)
            nv = deg - g * 16
            sc = jnp.where(lanes < nv, sc, NEG)
            m_new = jnp.maximum(m_v, jnp.full((16,), jnp.max(sc)))
            scale = jnp.exp(m_v - m_new)
            ex = jnp.exp(sc - m_new)
            denom_v = denom_v * scale + ex
            ejs = [jnp.full((16,), ex[j]) for j in range(16)]
            pltpu.make_async_copy(h_hbm.at[lanes], rows_v.at[buf],
                                  sem.at[buf]).wait()
            new_acc = [acc[k] * scale for k in range(16)]
            for j in range(16):
                for k in range(16):
                    new_acc[k] = (new_acc[k]
                                  + ejs[j] * rows_v[buf, j, pl.ds(k * 16, 16)])
            cb = issue(buf, la_base, cb)
            la_row, la_base = advance(la_row, la_base)
            return (q + 1, cb, la_row, la_base, m_new, denom_v,
                    tuple(new_acc))

        z = jnp.zeros((16,), jnp.float32)
        init = (q, cb, la_row, la_base, jnp.full((16,), NEG, jnp.float32),
                z, tuple(z for _ in range(16)))
        q, cb, la_row, la_base, m_v, denom_v, acc = lax.fori_loop(
            0, ngrp, grp_body, init)
        inv = 1.0 / (jnp.full((16,), jnp.sum(denom_v)) + 1e-16)

        ob = ro % NB

        @pl.when(ro >= NB)
        def _():
            pltpu.make_async_copy(orow_v.at[ob], out_hbm.at[r],
                                  osem.at[ob]).wait()

        for k in range(16):
            orow_v[ob, pl.ds(k * 16, 16)] = (acc[k] * inv
                                             + bias_v[pl.ds(k * 16, 16)])
        pltpu.async_copy(orow_v.at[ob], out_hbm.at[r], osem.at[ob])
        return (q, cb, la_row, la_base, ro + 1)

    q, cb, la_row, la_base, ro = lax.fori_loop(
        rlo, rhi, row_body,
        (jnp.int32(0), cb, la_row, la_base, jnp.int32(0)))

    for i in range(NB):
        pltpu.make_async_copy(h_hbm.at[lanes], rows_v.at[i],
                              sem.at[i]).wait()

        @pl.when(ro > i)
        def _(i=i):
            pltpu.make_async_copy(orow_v.at[i], out_hbm.at[0],
                                  osem.at[i]).wait()


_edge_kernel = pl.kernel(
    _gat_edge_body,
    out_type=jax.ShapeDtypeStruct((NPAD, F), jnp.float32),
    mesh=plsc.VectorSubcoreMesh(core_axis_name="c", subcore_axis_name="s"),
    compiler_params=pltpu.CompilerParams(needs_layout_passes=False),
    scratch_types=[
        pltpu.VMEM((NPAD + 8,), jnp.int32),
        pltpu.VMEM((NPAD + 16,), jnp.float32),
        pltpu.VMEM((NPAD,), jnp.float32),
        pltpu.VMEM((F,), jnp.float32),
        pltpu.VMEM((CSZ,), jnp.int32),
        pltpu.VMEM((NB, 16, F), jnp.float32),
        pltpu.VMEM((NB, 16), jnp.float32),
        pltpu.VMEM((NB, F), jnp.float32),
        pltpu.SemaphoreType.DMA((NB,)),
        pltpu.SemaphoreType.DMA((NB,)),
    ],
)


def _gat_layer(x_pad, W, al, ar, bias, rowp_pad, col_pad):
    h, el, er = _dense_stage(x_pad, W, al, ar)
    return _edge_kernel(rowp_pad, col_pad, el, er, h, bias)


@jax.jit
def _impl(row_ptr, col_idx, input_feature, W1, attn_l1, attn_r1, bias1,
          W2, attn_l2, attn_r2, bias2):
    rowp = row_ptr.astype(jnp.int32)
    col = col_idx.astype(jnp.int32)
    rowp_pad = jnp.concatenate(
        [rowp, jnp.full((NPAD + 8 - (N + 1),), E, jnp.int32)])
    col_pad = jnp.concatenate([col, jnp.zeros((COLPAD - E,), jnp.int32)])
    x_pad = jnp.pad(input_feature, ((0, NPAD - N), (0, 0)))
    x1 = _gat_layer(x_pad, W1, attn_l1, attn_r1, bias1, rowp_pad, col_pad)
    x2 = _gat_layer(x1, W2, attn_l2, attn_r2, bias2, rowp_pad, col_pad)
    return (x2[:N], jnp.float32(0.0))


def kernel(row_ptr, col_idx, input_feature, W1, attn_l1, attn_r1, bias1,
           W2, attn_l2, attn_r2, bias2):
    return _impl(row_ptr, col_idx, input_feature, W1, attn_l1, attn_r1,
                 bias1, W2, attn_l2, attn_r2, bias2)


# speedup vs baseline: 1.1923x; 1.1923x over previous
import functools

import jax
import jax.numpy as jnp
from jax import lax
from jax.experimental import pallas as pl
from jax.experimental.pallas import tpu as pltpu
from jax.experimental.pallas import tpu_sc as plsc

N = 10000
E = 160000
F = 256
SLOPE = 0.2
NC, NS, L = 2, 16, 16
NW = NC * NS
NPAD = 10240
NEG = -1e30
NB = 8
CSZ = 2048
EPW = E // NW
COLPAD = E + CSZ + 8


def _mm_body(x_ref, w_ref, al_ref, ar_ref, h_ref, el_ref, er_ref):
    h = jnp.dot(x_ref[...], w_ref[...], preferred_element_type=jnp.float32)
    h_ref[...] = h
    el_ref[...] = jnp.dot(h, al_ref[...], preferred_element_type=jnp.float32)
    er_ref[...] = jnp.dot(h, ar_ref[...], preferred_element_type=jnp.float32)


def _dense_stage(x, W, al, ar):
    BM = 256
    h, el, er = pl.pallas_call(
        _mm_body,
        grid=(NPAD // BM,),
        in_specs=[
            pl.BlockSpec((BM, F), lambda i: (i, 0)),
            pl.BlockSpec((F, F), lambda i: (0, 0)),
            pl.BlockSpec((F, 1), lambda i: (0, 0)),
            pl.BlockSpec((F, 1), lambda i: (0, 0)),
        ],
        out_specs=[
            pl.BlockSpec((BM, F), lambda i: (i, 0)),
            pl.BlockSpec((BM, 1), lambda i: (i, 0)),
            pl.BlockSpec((BM, 1), lambda i: (i, 0)),
        ],
        out_shape=[
            jax.ShapeDtypeStruct((NPAD, F), jnp.float32),
            jax.ShapeDtypeStruct((NPAD, 1), jnp.float32),
            jax.ShapeDtypeStruct((NPAD, 1), jnp.float32),
        ],
    )(x, W, al.reshape(F, 1), ar.reshape(F, 1))
    return h, el.reshape(NPAD), er.reshape(NPAD)


def _gat_edge_body(rowp_hbm, cidx_hbm, el_hbm, er_hbm, h_hbm, bias_hbm,
                   out_hbm, rowp_v, el_v, er_v, bias_v, cidx_v, rows_v,
                   erring_v, orow_v, sem, osem):
    c = lax.axis_index("c")
    s = lax.axis_index("s")
    wid = s * NC + c

    pltpu.sync_copy(rowp_hbm, rowp_v)
    pltpu.sync_copy(el_hbm, el_v.at[pl.ds(0, NPAD)])
    pltpu.sync_copy(er_hbm, er_v)
    pltpu.sync_copy(bias_hbm, bias_v)

    lanes = lax.iota(jnp.int32, 16)

    def rp_at(i):
        return rowp_v[pl.ds(i, 16)][0]

    def bsearch(target):
        def bb(_, lohi):
            lo, hi = lohi
            run = lo < hi
            mid = (lo + hi) // 2
            go_hi = rp_at(mid) >= target
            nlo = jnp.where(go_hi, lo, mid + 1)
            nhi = jnp.where(go_hi, mid, hi)
            return (jnp.where(run, nlo, lo), jnp.where(run, nhi, hi))
        lo, _ = lax.fori_loop(0, 15, bb, (jnp.int32(0), jnp.int32(N)))
        return lo

    rlo = bsearch(wid * EPW)
    rhi = jnp.where(wid == NW - 1, jnp.int32(N), bsearch((wid + 1) * EPW))

    def skip_empty(row, nb):
        def cond(carry):
            return carry[1] != 0
        def body(carry):
            r, _ = carry
            adv = (rp_at(r + 1) <= nb) & (r + 1 < rhi)
            return (r + jnp.where(adv, 1, 0), adv.astype(jnp.int32))
        r, _ = lax.while_loop(cond, body, (row, jnp.int32(1)))
        return r

    def issue(buf, la_base, cb):
        need = la_base + 16 > cb + CSZ
        nstart = pl.multiple_of((la_base // 8) * 8, 8)
        ncb = jnp.where(need, nstart, cb)

        @pl.when(need)
        def _():
            pltpu.sync_copy(cidx_hbm.at[pl.ds(nstart, CSZ)], cidx_v)

        nidx = cidx_v[pl.ds(la_base - ncb, 16)]
        erring_v[buf, :] = plsc.load_gather(er_v, [nidx])
        pltpu.async_copy(h_hbm.at[nidx], rows_v.at[buf], sem.at[buf])
        return ncb

    def advance(la_row, la_base):
        e1la = rp_at(la_row + 1)
        nb = jnp.minimum(la_base + 16, e1la)
        nrow = jnp.where(nb < e1la, la_row, skip_empty(la_row, nb))
        return nrow, nb

    la_base = rp_at(rlo)
    la_row = skip_empty(rlo, la_base)
    cb = jnp.int32(-(CSZ + 64))
    for i in range(NB):
        cb = issue(jnp.int32(i), la_base, cb)
        la_row, la_base = advance(la_row, la_base)

    def row_body(r, rcarry):
        q, cb, la_row, la_base, ro = rcarry
        rp = rowp_v[pl.ds(r, 16)]
        e0 = rp[0]
        e1 = rp[1]
        deg = e1 - e0
        ngrp = (deg + 15) // 16
        el_r = jnp.full((16,), el_v[pl.ds(r, 16)][0], jnp.float32)

        def grp_body(g, carry):
            q, cb, la_row, la_base, m_v, denom_v, acc = carry
            buf = q % NB
            erg = erring_v[buf, :]
            sc = el_r + erg
            sc = jnp.where(sc > 0, sc, SLOPE * sc)
            nv = deg - g * 16
            sc = jnp.where(lanes < nv, sc, NEG)
            m_new = jnp.maximum(m_v, jnp.full((16,), jnp.max(sc)))
            scale = jnp.exp(m_v - m_new)
            ex = jnp.exp(sc - m_new)
            denom_v = denom_v * scale + ex
            pltpu.make_async_copy(h_hbm.at[lanes], rows_v.at[buf],
                                  sem.at[buf]).wait()
            new_acc = []
            for k in range(16):
                a = acc[k] * scale
                for j in range(16):
                    ejv = jnp.full((16,), ex[j])
                    a = a + ejv * rows_v[buf, j, pl.ds(k * 16, 16)]
                new_acc.append(a)
            cb = issue(buf, la_base, cb)
            la_row, la_base = advance(la_row, la_base)
            return (q + 1, cb, la_row, la_base, m_new, denom_v,
                    tuple(new_acc))

        z = jnp.zeros((16,), jnp.float32)
        init = (q, cb, la_row, la_base, jnp.full((16,), NEG, jnp.float32),
                z, tuple(z for _ in range(16)))
        q, cb, la_row, la_base, m_v, denom_v, acc = lax.fori_loop(
            0, ngrp, grp_body, init)
        inv = 1.0 / (jnp.full((16,), jnp.sum(denom_v)) + 1e-16)

        ob = ro % NB

        @pl.when(ro >= NB)
        def _():
            pltpu.make_async_copy(orow_v.at[ob], out_hbm.at[r],
                                  osem.at[ob]).wait()

        for k in range(16):
            orow_v[ob, pl.ds(k * 16, 16)] = (acc[k] * inv
                                             + bias_v[pl.ds(k * 16, 16)])
        pltpu.async_copy(orow_v.at[ob], out_hbm.at[r], osem.at[ob])
        return (q, cb, la_row, la_base, ro + 1)

    q, cb, la_row, la_base, ro = lax.fori_loop(
        rlo, rhi, row_body,
        (jnp.int32(0), cb, la_row, la_base, jnp.int32(0)))

    for i in range(NB):
        pltpu.make_async_copy(h_hbm.at[lanes], rows_v.at[i],
                              sem.at[i]).wait()

        @pl.when(ro > i)
        def _(i=i):
            pltpu.make_async_copy(orow_v.at[i], out_hbm.at[0],
                                  osem.at[i]).wait()


_edge_kernel = pl.kernel(
    _gat_edge_body,
    out_type=jax.ShapeDtypeStruct((NPAD, F), jnp.float32),
    mesh=plsc.VectorSubcoreMesh(core_axis_name="c", subcore_axis_name="s"),
    compiler_params=pltpu.CompilerParams(needs_layout_passes=False),
    scratch_types=[
        pltpu.VMEM((NPAD + 8,), jnp.int32),
        pltpu.VMEM((NPAD + 16,), jnp.float32),
        pltpu.VMEM((NPAD,), jnp.float32),
        pltpu.VMEM((F,), jnp.float32),
        pltpu.VMEM((CSZ,), jnp.int32),
        pltpu.VMEM((NB, 16, F), jnp.float32),
        pltpu.VMEM((NB, 16), jnp.float32),
        pltpu.VMEM((NB, F), jnp.float32),
        pltpu.SemaphoreType.DMA((NB,)),
        pltpu.SemaphoreType.DMA((NB,)),
    ],
)


def _gat_layer(x_pad, W, al, ar, bias, rowp_pad, col_pad):
    h, el, er = _dense_stage(x_pad, W, al, ar)
    return _edge_kernel(rowp_pad, col_pad, el, er, h, bias)


@jax.jit
def _impl(row_ptr, col_idx, input_feature, W1, attn_l1, attn_r1, bias1,
          W2, attn_l2, attn_r2, bias2):
    rowp = row_ptr.astype(jnp.int32)
    col = col_idx.astype(jnp.int32)
    rowp_pad = jnp.concatenate(
        [rowp, jnp.full((NPAD + 8 - (N + 1),), E, jnp.int32)])
    col_pad = jnp.concatenate([col, jnp.zeros((COLPAD - E,), jnp.int32)])
    x_pad = jnp.pad(input_feature, ((0, NPAD - N), (0, 0)))
    x1 = _gat_layer(x_pad, W1, attn_l1, attn_r1, bias1, rowp_pad, col_pad)
    x2 = _gat_layer(x1, W2, attn_l2, attn_r2, bias2, rowp_pad, col_pad)
    return (x2[:N], jnp.float32(0.0))


def kernel(row_ptr, col_idx, input_feature, W1, attn_l1, attn_r1, bias1,
           W2, attn_l2, attn_r2, bias2):
    return _impl(row_ptr, col_idx, input_feature, W1, attn_l1, attn_r1,
                 bias1, W2, attn_l2, attn_r2, bias2)


# speedup vs baseline: 1.2278x; 1.0298x over previous
import functools

import jax
import jax.numpy as jnp
from jax import lax
from jax.experimental import pallas as pl
from jax.experimental.pallas import tpu as pltpu
from jax.experimental.pallas import tpu_sc as plsc

N = 10000
E = 160000
F = 256
SLOPE = 0.2
NC, NS, L = 2, 16, 16
NW = NC * NS
NPAD = 10240
NEG = -1e30
NB = 4
CSZ = 2048
EPW = E // NW
COLPAD = E + CSZ + 8


def _mm_body(x_ref, w_ref, al_ref, ar_ref, h_ref, el_ref, er_ref):
    h = jnp.dot(x_ref[...], w_ref[...], preferred_element_type=jnp.float32)
    h_ref[...] = h
    el_ref[...] = jnp.dot(h, al_ref[...], preferred_element_type=jnp.float32)
    er_ref[...] = jnp.dot(h, ar_ref[...], preferred_element_type=jnp.float32)


def _dense_stage(x, W, al, ar):
    BM = 256
    h, el, er = pl.pallas_call(
        _mm_body,
        grid=(NPAD // BM,),
        in_specs=[
            pl.BlockSpec((BM, F), lambda i: (i, 0)),
            pl.BlockSpec((F, F), lambda i: (0, 0)),
            pl.BlockSpec((F, 1), lambda i: (0, 0)),
            pl.BlockSpec((F, 1), lambda i: (0, 0)),
        ],
        out_specs=[
            pl.BlockSpec((BM, F), lambda i: (i, 0)),
            pl.BlockSpec((BM, 1), lambda i: (i, 0)),
            pl.BlockSpec((BM, 1), lambda i: (i, 0)),
        ],
        out_shape=[
            jax.ShapeDtypeStruct((NPAD, F), jnp.float32),
            jax.ShapeDtypeStruct((NPAD, 1), jnp.float32),
            jax.ShapeDtypeStruct((NPAD, 1), jnp.float32),
        ],
    )(x, W, al.reshape(F, 1), ar.reshape(F, 1))
    return h, el.reshape(NPAD), er.reshape(NPAD)


def _gat_edge_body(rowp_hbm, cidx_hbm, el_hbm, er_hbm, h_hbm, bias_hbm,
                   out_hbm, rowp_v, el_v, er_v, bias_v, cidx_v, rows_v,
                   erring_v, orow_v, sem, osem):
    c = lax.axis_index("c")
    s = lax.axis_index("s")
    wid = s * NC + c

    pltpu.sync_copy(rowp_hbm, rowp_v)
    pltpu.sync_copy(el_hbm, el_v.at[pl.ds(0, NPAD)])
    pltpu.sync_copy(er_hbm, er_v)
    pltpu.sync_copy(bias_hbm, bias_v)

    lanes = lax.iota(jnp.int32, 16)

    def rp_at(i):
        return rowp_v[pl.ds(i, 16)][0]

    def bsearch(target):
        def bb(_, lohi):
            lo, hi = lohi
            run = lo < hi
            mid = (lo + hi) // 2
            go_hi = rp_at(mid) >= target
            nlo = jnp.where(go_hi, lo, mid + 1)
            nhi = jnp.where(go_hi, mid, hi)
            return (jnp.where(run, nlo, lo), jnp.where(run, nhi, hi))
        lo, _ = lax.fori_loop(0, 15, bb, (jnp.int32(0), jnp.int32(N)))
        return lo

    rlo = bsearch(wid * EPW)
    rhi = jnp.where(wid == NW - 1, jnp.int32(N), bsearch((wid + 1) * EPW))

    def skip_empty(row, nb):
        def cond(carry):
            return carry[1] != 0
        def body(carry):
            r, _ = carry
            adv = (rp_at(r + 1) <= nb) & (r + 1 < rhi)
            return (r + jnp.where(adv, 1, 0), adv.astype(jnp.int32))
        r, _ = lax.while_loop(cond, body, (row, jnp.int32(1)))
        return r

    def issue(buf, la_base, cb):
        need = la_base + 16 > cb + CSZ
        nstart = pl.multiple_of((la_base // 8) * 8, 8)
        ncb = jnp.where(need, nstart, cb)

        @pl.when(need)
        def _():
            pltpu.sync_copy(cidx_hbm.at[pl.ds(nstart, CSZ)], cidx_v)

        nidx = cidx_v[pl.ds(la_base - ncb, 16)]
        erring_v[buf, :] = plsc.load_gather(er_v, [nidx])
        pltpu.async_copy(h_hbm.at[nidx], rows_v.at[buf], sem.at[buf])
        return ncb

    def advance(la_row, la_base):
        nb = jnp.minimum(la_base + 16, rp_at(la_row + 1))
        return skip_empty(la_row, nb), nb

    la_base = rp_at(rlo)
    la_row = skip_empty(rlo, la_base)
    cb = jnp.int32(-(CSZ + 64))
    for i in range(NB):
        cb = issue(jnp.int32(i), la_base, cb)
        la_row, la_base = advance(la_row, la_base)

    def row_body(r, rcarry):
        q, cb, la_row, la_base, ro = rcarry
        rp = rowp_v[pl.ds(r, 16)]
        e0 = rp[0]
        e1 = rp[1]
        deg = e1 - e0
        ngrp = (deg + 15) // 16
        el_r = jnp.full((16,), el_v[pl.ds(r, 16)][0], jnp.float32)

        def grp_body(g, carry):
            q, cb, la_row, la_base, m_v, denom_v, acc = carry
            buf = q % NB
            pltpu.make_async_copy(h_hbm.at[lanes], rows_v.at[buf],
                                  sem.at[buf]).wait()
            erg = erring_v[buf, :]
            sc = el_r + erg
            sc = jnp.where(sc > 0, sc, SLOPE * sc)
            nv = deg - g * 16
            sc = jnp.where(lanes < nv, sc, NEG)
            m_new = jnp.maximum(m_v, jnp.full((16,), jnp.max(sc)))
            scale = jnp.exp(m_v - m_new)
            ex = jnp.exp(sc - m_new)
            denom_v = denom_v * scale + ex
            new_acc = []
            for k in range(16):
                a = acc[k] * scale
                for j in range(16):
                    ejv = jnp.full((16,), ex[j])
                    a = a + ejv * rows_v[buf, j, pl.ds(k * 16, 16)]
                new_acc.append(a)
            cb = issue(buf, la_base, cb)
            la_row, la_base = advance(la_row, la_base)
            return (q + 1, cb, la_row, la_base, m_new, denom_v,
                    tuple(new_acc))

        z = jnp.zeros((16,), jnp.float32)
        init = (q, cb, la_row, la_base, jnp.full((16,), NEG, jnp.float32),
                z, tuple(z for _ in range(16)))
        q, cb, la_row, la_base, m_v, denom_v, acc = lax.fori_loop(
            0, ngrp, grp_body, init)
        inv = 1.0 / (jnp.full((16,), jnp.sum(denom_v)) + 1e-16)

        ob = ro % NB

        @pl.when(ro >= NB)
        def _():
            pltpu.make_async_copy(orow_v.at[ob], out_hbm.at[r],
                                  osem.at[ob]).wait()

        for k in range(16):
            orow_v[ob, pl.ds(k * 16, 16)] = (acc[k] * inv
                                             + bias_v[pl.ds(k * 16, 16)])
        pltpu.async_copy(orow_v.at[ob], out_hbm.at[r], osem.at[ob])
        return (q, cb, la_row, la_base, ro + 1)

    q, cb, la_row, la_base, ro = lax.fori_loop(
        rlo, rhi, row_body,
        (jnp.int32(0), cb, la_row, la_base, jnp.int32(0)))

    for i in range(NB):
        pltpu.make_async_copy(h_hbm.at[lanes], rows_v.at[i],
                              sem.at[i]).wait()

        @pl.when(ro > i)
        def _(i=i):
            pltpu.make_async_copy(orow_v.at[i], out_hbm.at[0],
                                  osem.at[i]).wait()


_edge_kernel = pl.kernel(
    _gat_edge_body,
    out_type=jax.ShapeDtypeStruct((NPAD, F), jnp.float32),
    mesh=plsc.VectorSubcoreMesh(core_axis_name="c", subcore_axis_name="s"),
    compiler_params=pltpu.CompilerParams(needs_layout_passes=False),
    scratch_types=[
        pltpu.VMEM((NPAD + 8,), jnp.int32),
        pltpu.VMEM((NPAD + 16,), jnp.float32),
        pltpu.VMEM((NPAD,), jnp.float32),
        pltpu.VMEM((F,), jnp.float32),
        pltpu.VMEM((CSZ,), jnp.int32),
        pltpu.VMEM((NB, 16, F), jnp.float32),
        pltpu.VMEM((NB, 16), jnp.float32),
        pltpu.VMEM((NB, F), jnp.float32),
        pltpu.SemaphoreType.DMA((NB,)),
        pltpu.SemaphoreType.DMA((NB,)),
    ],
)


def _gat_layer(x_pad, W, al, ar, bias, rowp_pad, col_pad):
    h, el, er = _dense_stage(x_pad, W, al, ar)
    return _edge_kernel(rowp_pad, col_pad, el, er, h, bias)


@jax.jit
def _impl(row_ptr, col_idx, input_feature, W1, attn_l1, attn_r1, bias1,
          W2, attn_l2, attn_r2, bias2):
    rowp = row_ptr.astype(jnp.int32)
    col = col_idx.astype(jnp.int32)
    rowp_pad = jnp.concatenate(
        [rowp, jnp.full((NPAD + 8 - (N + 1),), E, jnp.int32)])
    col_pad = jnp.concatenate([col, jnp.zeros((COLPAD - E,), jnp.int32)])
    x_pad = jnp.pad(input_feature, ((0, NPAD - N), (0, 0)))
    x1 = _gat_layer(x_pad, W1, attn_l1, attn_r1, bias1, rowp_pad, col_pad)
    x2 = _gat_layer(x1, W2, attn_l2, attn_r2, bias2, rowp_pad, col_pad)
    return (x2[:N], jnp.float32(0.0))


def kernel(row_ptr, col_idx, input_feature, W1, attn_l1, attn_r1, bias1,
           W2, attn_l2, attn_r2, bias2):
    return _impl(row_ptr, col_idx, input_feature, W1, attn_l1, attn_r1,
                 bias1, W2, attn_l2, attn_r2, bias2)
